# baseline (speedup 1.0000x reference)
import functools

import jax
import jax.numpy as jnp
import numpy as np
from jax import lax
from jax.experimental import pallas as pl
from jax.experimental.pallas import tpu as pltpu
from jax.experimental.pallas import tpu_sc as plsc

DMIN, DMAX, STEP = 0.0, 8.0, 0.25
VAR = STEP
NFILT = 33
NC, NS = 2, 16
SUBS_PER_CASE = 8


def _sc_dist2(coords_raw, nbr_raw, lat_pad, B, N, M):
    NM = N * M
    EPW = NM // SUBS_PER_CASE
    APW = N // SUBS_PER_CASE
    NCH = 5
    ACH = N // NCH
    ECH = EPW // NCH
    ACH_E = ECH // M
    assert ACH % 16 == 0 and ECH % M == 0
    mesh = plsc.VectorSubcoreMesh(
        core_axis_name="c", subcore_axis_name="s", num_cores=NC, num_subcores=NS
    )

    @functools.partial(
        pl.kernel,
        mesh=mesh,
        compiler_params=pltpu.CompilerParams(needs_layout_passes=False),
        out_type=jax.ShapeDtypeStruct((2 * B * NM,), jnp.float32),
        scratch_types=[
            pltpu.VMEM((N,), jnp.float32),
            pltpu.VMEM((N,), jnp.float32),
            pltpu.VMEM((N,), jnp.float32),
            pltpu.VMEM((6 * ACH,), jnp.float32),
            pltpu.VMEM((2 * ECH,), jnp.int32),
            pltpu.VMEM((EPW,), jnp.float32),
            pltpu.VMEM((48,), jnp.float32),
        ],
    )
    def k(coords_hbm, nbr_hbm, lat_hbm, out, xv, yv, zv, craw, nraw, ov, latv):
        c = lax.axis_index("c")
        s = lax.axis_index("s")
        b = s // SUBS_PER_CASE
        sub = s % SUBS_PER_CASE
        base = sub * EPW
        atoms0 = sub * APW
        case = c * B + b

        pltpu.sync_copy(lat_hbm.at[pl.ds(case * 48, 48)], latv)
        lxv = latv[pl.ds(0, 16)]
        lyv = latv[pl.ds(16, 16)]
        lzv = latv[pl.ds(32, 16)]
        hxv = lxv * 0.5
        hyv = lyv * 0.5
        hzv = lzv * 0.5

        lane = lax.iota(jnp.int32, 16)

        for ch in range(NCH):
            pltpu.sync_copy(
                coords_hbm.at[pl.ds(b * N * 6 + ch * ACH * 6, ACH * 6)], craw
            )

            def extract(i, _):
                src = i * 16 * 6 + lane * 6
                dst = ch * ACH + i * 16
                xv[pl.ds(dst, 16)] = plsc.load_gather(craw, [src + c])
                yv[pl.ds(dst, 16)] = plsc.load_gather(craw, [src + 2 + c])
                zv[pl.ds(dst, 16)] = plsc.load_gather(craw, [src + 4 + c])
                return 0

            lax.fori_loop(0, ACH // 16, extract, 0, unroll=4)

        for ch in range(NCH):
            pltpu.sync_copy(
                nbr_hbm.at[pl.ds((b * NM + base + ch * ECH) * 2, 2 * ECH)], nraw
            )

            def body(a, _):
                gidx = jnp.full((16,), atoms0 + ch * ACH_E + a, jnp.int32)
                ax = plsc.load_gather(xv, [gidx])
                ay = plsc.load_gather(yv, [gidx])
                az = plsc.load_gather(zv, [gidx])
                for h in range(M // 16):
                    loff = a * M + h * 16
                    idx = plsc.load_gather(nraw, [2 * (loff + lane) + c])
                    dx = jnp.abs(ax - plsc.load_gather(xv, [idx]))
                    dy = jnp.abs(ay - plsc.load_gather(yv, [idx]))
                    dz = jnp.abs(az - plsc.load_gather(zv, [idx]))
                    dx = jnp.where(dx > hxv, dx - lxv, dx)
                    dy = jnp.where(dy > hyv, dy - lyv, dy)
                    dz = jnp.where(dz > hzv, dz - lzv, dz)
                    ov[pl.ds(ch * ECH + loff, 16)] = dx * dx + dy * dy + dz * dz
                return 0

            lax.fori_loop(0, ACH_E, body, 0, unroll=2)

        pltpu.sync_copy(ov, out.at[pl.ds(case * NM + base, EPW)])

    return k(coords_raw, nbr_raw, lat_pad)


def _tc_expand(d2_all, stack, E):
    EB = E // 128
    RQ = 40 if EB % 40 == 0 else 8
    assert EB % RQ == 0
    base_blk = stack * (EB // RQ)
    inv_var2 = 1.0 / (VAR * VAR)
    d2_rows = d2_all.reshape(2 * EB, 128)

    def body(d2_ref, o_ref):
        d = jnp.sqrt(d2_ref[...])
        dt = jnp.transpose(d)
        f = lax.broadcasted_iota(jnp.int32, (128, NFILT), 1).astype(jnp.float32) * STEP
        for q in range(RQ):
            dq = jnp.broadcast_to(dt[:, q:q + 1], (128, NFILT))
            diff = dq - f
            o_ref[q * 128:(q + 1) * 128, :] = jnp.exp(diff * diff * (-inv_var2))

    return pl.pallas_call(
        body,
        grid=(EB // RQ,),
        in_specs=[pl.BlockSpec((RQ, 128), lambda i: (i + base_blk, 0))],
        out_specs=pl.BlockSpec((RQ * 128, NFILT), lambda i: (i, 0)),
        out_shape=jax.ShapeDtypeStruct((E, NFILT), jnp.float32),
    )(d2_rows)


def kernel(stacked_coords, stacked_lattices, stacked_nbr_lists):
    B, N = stacked_coords.shape[0], stacked_coords.shape[1]
    M = stacked_nbr_lists.shape[2]

    nbr1 = stacked_nbr_lists[..., 0]
    nbr2 = stacked_nbr_lists[..., 1]
    lat_t = jnp.transpose(stacked_lattices, (2, 0, 1))
    lat_pad = jnp.broadcast_to(lat_t[..., None], (2, B, 3, 16)).reshape(-1)

    d2_all = _sc_dist2(
        stacked_coords.reshape(-1), stacked_nbr_lists.reshape(-1), lat_pad, B, N, M
    )

    E = B * N * M
    bond_fea_1 = _tc_expand(d2_all, 0, E).reshape(B, N, M, NFILT)
    bond_fea_2 = _tc_expand(d2_all, 1, E).reshape(B, N, M, NFILT)
    return (nbr1, bond_fea_1, nbr2, bond_fea_2)


# speedup vs baseline: 8.7907x; 8.7907x over previous
import functools

import jax
import jax.numpy as jnp
from jax import lax
from jax.experimental import pallas as pl
from jax.experimental.pallas import tpu as pltpu
from jax.experimental.pallas import tpu_sc as plsc

DMIN, DMAX, STEP = 0.0, 8.0, 0.25
VAR = STEP
NFILT = 33
NC, NS = 2, 16
SUBS_PER_CASE = 8


def _sc_dist2(coords_p, nbr_p, lat_pad, B, N, M, NP):
    NT = NP // 128
    NTW = NT // SUBS_PER_CASE
    LW = NTW * 128
    MT = M // 8
    CASE_STRIDE = MT * NT * 1024
    MT_STRIDE = NT * 1024
    mesh = plsc.VectorSubcoreMesh(
        core_axis_name="c", subcore_axis_name="s", num_cores=NC, num_subcores=NS
    )

    @functools.partial(
        pl.kernel,
        mesh=mesh,
        compiler_params=pltpu.CompilerParams(needs_layout_passes=False),
        out_type=jax.ShapeDtypeStruct((2 * B * CASE_STRIDE,), jnp.float32),
        scratch_types=[
            pltpu.VMEM((NP,), jnp.float32),
            pltpu.VMEM((NP,), jnp.float32),
            pltpu.VMEM((NP,), jnp.float32),
            pltpu.VMEM((M * LW,), jnp.int32),
            pltpu.VMEM((MT * NTW * 1024,), jnp.float32),
            pltpu.VMEM((48,), jnp.float32),
            pltpu.SemaphoreType.DMA,
        ],
    )
    def k(coords_hbm, nbr_hbm, lat_hbm, out, xv, yv, zv, nb, ov, latv, sem):
        c = lax.axis_index("c")
        s = lax.axis_index("s")
        b = s // SUBS_PER_CASE
        sub = s % SUBS_PER_CASE
        nt0 = sub * NTW
        case = c * B + b

        pltpu.sync_copy(lat_hbm.at[pl.ds(case * 48, 48)], latv)
        lxv = latv[pl.ds(0, 16)]
        lyv = latv[pl.ds(16, 16)]
        lzv = latv[pl.ds(32, 16)]
        hxv = lxv * 0.5
        hyv = lyv * 0.5
        hzv = lzv * 0.5

        cbase = (b * 3) * 2 + c
        pltpu.sync_copy(coords_hbm.at[pl.ds((cbase + 0) * NP, NP)], xv)
        pltpu.sync_copy(coords_hbm.at[pl.ds((cbase + 2) * NP, NP)], yv)
        pltpu.sync_copy(coords_hbm.at[pl.ds((cbase + 4) * NP, NP)], zv)

        copies = []
        for m in range(M):
            off = ((b * M + m) * 2 + c) * NP + nt0 * 128
            copies.append(
                pltpu.make_async_copy(
                    nbr_hbm.at[pl.ds(off, LW)], nb.at[pl.ds(m * LW, LW)], sem
                )
            )
        for cp in copies:
            cp.start()
        for cp in copies:
            cp.wait()

        def mbody(m, _):
            obase = (m // 8) * (NTW * 1024) + (m % 8) * 128

            def ntbody(t, _):
                for j in range(8):
                    p = t * 128 + j * 16
                    idx = nb[pl.ds(m * LW + p, 16)]
                    ax = xv[pl.ds(nt0 * 128 + p, 16)]
                    ay = yv[pl.ds(nt0 * 128 + p, 16)]
                    az = zv[pl.ds(nt0 * 128 + p, 16)]
                    dx = jnp.abs(ax - plsc.load_gather(xv, [idx]))
                    dy = jnp.abs(ay - plsc.load_gather(yv, [idx]))
                    dz = jnp.abs(az - plsc.load_gather(zv, [idx]))
                    dx = jnp.where(dx > hxv, dx - lxv, dx)
                    dy = jnp.where(dy > hyv, dy - lyv, dy)
                    dz = jnp.where(dz > hzv, dz - lzv, dz)
                    ov[pl.ds(obase + t * 1024 + j * 16, 16)] = (
                        dx * dx + dy * dy + dz * dz
                    )
                return 0

            lax.fori_loop(0, NTW, ntbody, 0)
            return 0

        lax.fori_loop(0, M, mbody, 0)

        for mt in range(MT):
            pltpu.sync_copy(
                ov.at[pl.ds(mt * NTW * 1024, NTW * 1024)],
                out.at[pl.ds(case * CASE_STRIDE + mt * MT_STRIDE + nt0 * 1024,
                             NTW * 1024)],
            )

    return k(coords_p, nbr_p, lat_pad)


def _tc_expand(d2v, stack, B, N, M, NP):
    MT = M // 8
    NT = NP // 128
    NTB = 10 if NT % 10 == 0 else (8 if NT % 8 == 0 else 1)
    NL = NTB * 128
    nblk = NT // NTB
    inv_var2 = 1.0 / (VAR * VAR)

    def body(d2_ref, o_ref):
        d = jnp.sqrt(d2_ref[...])
        for mt in range(MT):
            for t in range(NTB):
                dv = d[0, mt, t]
                for k in range(NFILT):
                    diff = dv - (k * STEP)
                    o_ref[0, k, mt * 8:(mt + 1) * 8, t * 128:(t + 1) * 128] = (
                        jnp.exp(diff * diff * (-inv_var2))
                    )

    return pl.pallas_call(
        body,
        grid=(B, nblk),
        in_specs=[
            pl.BlockSpec((1, MT, NTB, 8, 128),
                         lambda b, t: (stack * B + b, 0, t, 0, 0)),
        ],
        out_specs=pl.BlockSpec((1, NFILT, M, NL), lambda b, t: (b, 0, 0, t)),
        out_shape=jax.ShapeDtypeStruct((B, NFILT, M, N), jnp.float32),
    )(d2v)


def kernel(stacked_coords, stacked_lattices, stacked_nbr_lists):
    B, N = stacked_coords.shape[0], stacked_coords.shape[1]
    M = stacked_nbr_lists.shape[2]
    NP = -(-N // (SUBS_PER_CASE * 128)) * (SUBS_PER_CASE * 128)

    nbr1 = stacked_nbr_lists[..., 0]
    nbr2 = stacked_nbr_lists[..., 1]

    coords_p = jnp.pad(
        jnp.transpose(stacked_coords, (0, 2, 3, 1)),
        ((0, 0), (0, 0), (0, 0), (0, NP - N)),
    ).reshape(-1)
    nbr_p = jnp.pad(
        jnp.transpose(stacked_nbr_lists, (0, 2, 3, 1)),
        ((0, 0), (0, 0), (0, 0), (0, NP - N)),
    ).reshape(-1)
    lat_t = jnp.transpose(stacked_lattices, (2, 0, 1))
    lat_pad = jnp.broadcast_to(lat_t[..., None], (2, B, 3, 16)).reshape(-1)

    d2f = _sc_dist2(coords_p, nbr_p, lat_pad, B, N, M, NP)
    d2v = d2f.reshape(2 * B, M // 8, NP // 128, 8, 128)

    bf1 = _tc_expand(d2v, 0, B, N, M, NP)
    bf2 = _tc_expand(d2v, 1, B, N, M, NP)
    bond_fea_1 = jnp.transpose(bf1, (0, 3, 2, 1))
    bond_fea_2 = jnp.transpose(bf2, (0, 3, 2, 1))
    return (nbr1, bond_fea_1, nbr2, bond_fea_2)


# speedup vs baseline: 9.0695x; 1.0317x over previous
import functools

import jax
import jax.numpy as jnp
from jax import lax
from jax.experimental import pallas as pl
from jax.experimental.pallas import tpu as pltpu
from jax.experimental.pallas import tpu_sc as plsc

DMIN, DMAX, STEP = 0.0, 8.0, 0.25
VAR = STEP
NFILT = 33
NC, NS = 2, 16
SUBS_PER_CASE = 8


def _sc_dist2(coords_p, nbr_p, lat_pad, B, N, M, NP):
    NT = NP // 128
    NTW = NT // SUBS_PER_CASE
    LW = NTW * 128
    MT = M // 8
    CASE_STRIDE = MT * NT * 1024
    MT_STRIDE = NT * 1024
    mesh = plsc.VectorSubcoreMesh(
        core_axis_name="c", subcore_axis_name="s", num_cores=NC, num_subcores=NS
    )

    @functools.partial(
        pl.kernel,
        mesh=mesh,
        compiler_params=pltpu.CompilerParams(needs_layout_passes=False),
        out_type=jax.ShapeDtypeStruct((2 * B * CASE_STRIDE,), jnp.float32),
        scratch_types=[
            pltpu.VMEM((NP,), jnp.float32),
            pltpu.VMEM((NP,), jnp.float32),
            pltpu.VMEM((NP,), jnp.float32),
            pltpu.VMEM((M * LW,), jnp.int32),
            pltpu.VMEM((MT * NTW * 1024,), jnp.float32),
            pltpu.VMEM((48,), jnp.float32),
            pltpu.SemaphoreType.DMA,
        ],
    )
    def k(coords_hbm, nbr_hbm, lat_hbm, out, xv, yv, zv, nb, ov, latv, sem):
        c = lax.axis_index("c")
        s = lax.axis_index("s")
        b = s // SUBS_PER_CASE
        sub = s % SUBS_PER_CASE
        nt0 = sub * NTW
        case = c * B + b

        copies = []
        cbase = (b * 3) * 2 + c
        copies.append(pltpu.make_async_copy(
            coords_hbm.at[pl.ds((cbase + 0) * NP, NP)], xv, sem))
        copies.append(pltpu.make_async_copy(
            coords_hbm.at[pl.ds((cbase + 2) * NP, NP)], yv, sem))
        copies.append(pltpu.make_async_copy(
            coords_hbm.at[pl.ds((cbase + 4) * NP, NP)], zv, sem))
        for m in range(M):
            off = ((b * M + m) * 2 + c) * NP + nt0 * 128
            copies.append(pltpu.make_async_copy(
                nbr_hbm.at[pl.ds(off, LW)], nb.at[pl.ds(m * LW, LW)], sem))
        for cp in copies:
            cp.start()
        pltpu.sync_copy(lat_hbm.at[pl.ds(case * 48, 48)], latv)
        lxv = latv[pl.ds(0, 16)]
        lyv = latv[pl.ds(16, 16)]
        lzv = latv[pl.ds(32, 16)]
        for cp in copies:
            cp.wait()

        def pbody(p, _):
            base = p * 16
            ax = xv[pl.ds(nt0 * 128 + base, 16)]
            ay = yv[pl.ds(nt0 * 128 + base, 16)]
            az = zv[pl.ds(nt0 * 128 + base, 16)]
            t = base // 128
            j = base - t * 128
            odst = t * 1024 + j

            def mbody(m, _):
                idx = nb[pl.ds(m * LW + base, 16)]
                adx = jnp.abs(ax - plsc.load_gather(xv, [idx]))
                ady = jnp.abs(ay - plsc.load_gather(yv, [idx]))
                adz = jnp.abs(az - plsc.load_gather(zv, [idx]))
                ex = jnp.minimum(adx, lxv - adx)
                ey = jnp.minimum(ady, lyv - ady)
                ez = jnp.minimum(adz, lzv - adz)
                ov[pl.ds((m // 8) * (NTW * 1024) + (m % 8) * 128 + odst, 16)] = (
                    ex * ex + ey * ey + ez * ez
                )
                return 0

            lax.fori_loop(0, M, mbody, 0, unroll=4)
            return 0

        lax.fori_loop(0, LW // 16, pbody, 0)

        for mt in range(MT):
            pltpu.sync_copy(
                ov.at[pl.ds(mt * NTW * 1024, NTW * 1024)],
                out.at[pl.ds(case * CASE_STRIDE + mt * MT_STRIDE + nt0 * 1024,
                             NTW * 1024)],
            )

    return k(coords_p, nbr_p, lat_pad)


def _tc_expand(d2v, stack, B, N, M, NP):
    MT = M // 8
    NT = NP // 128
    NTB = 10 if NT % 10 == 0 else (8 if NT % 8 == 0 else 1)
    NL = NTB * 128
    nblk = NT // NTB
    inv_var2 = 1.0 / (VAR * VAR)

    def body(d2_ref, o_ref):
        d = jnp.sqrt(d2_ref[...])
        for mt in range(MT):
            for t in range(NTB):
                dv = d[0, mt, t]
                for k in range(NFILT):
                    diff = dv - (k * STEP)
                    o_ref[0, k, mt * 8:(mt + 1) * 8, t * 128:(t + 1) * 128] = (
                        jnp.exp(diff * diff * (-inv_var2))
                    )

    return pl.pallas_call(
        body,
        grid=(B, nblk),
        in_specs=[
            pl.BlockSpec((1, MT, NTB, 8, 128),
                         lambda b, t: (stack * B + b, 0, t, 0, 0)),
        ],
        out_specs=pl.BlockSpec((1, NFILT, M, NL), lambda b, t: (b, 0, 0, t)),
        out_shape=jax.ShapeDtypeStruct((B, NFILT, M, N), jnp.float32),
    )(d2v)


def kernel(stacked_coords, stacked_lattices, stacked_nbr_lists):
    B, N = stacked_coords.shape[0], stacked_coords.shape[1]
    M = stacked_nbr_lists.shape[2]
    NP = -(-N // (SUBS_PER_CASE * 128)) * (SUBS_PER_CASE * 128)

    nbr1 = stacked_nbr_lists[..., 0]
    nbr2 = stacked_nbr_lists[..., 1]

    coords_p = jnp.pad(
        jnp.transpose(stacked_coords, (0, 2, 3, 1)),
        ((0, 0), (0, 0), (0, 0), (0, NP - N)),
    ).reshape(-1)
    nbr_p = jnp.pad(
        jnp.transpose(stacked_nbr_lists, (0, 2, 3, 1)),
        ((0, 0), (0, 0), (0, 0), (0, NP - N)),
    ).reshape(-1)
    lat_t = jnp.transpose(stacked_lattices, (2, 0, 1))
    lat_pad = jnp.broadcast_to(lat_t[..., None], (2, B, 3, 16)).reshape(-1)

    d2f = _sc_dist2(coords_p, nbr_p, lat_pad, B, N, M, NP)
    d2v = d2f.reshape(2 * B, M // 8, NP // 128, 8, 128)

    bf1 = _tc_expand(d2v, 0, B, N, M, NP)
    bf2 = _tc_expand(d2v, 1, B, N, M, NP)
    bond_fea_1 = jnp.transpose(bf1, (0, 3, 2, 1))
    bond_fea_2 = jnp.transpose(bf2, (0, 3, 2, 1))
    return (nbr1, bond_fea_1, nbr2, bond_fea_2)


# speedup vs baseline: 11.9026x; 1.3124x over previous
import functools

import jax
import jax.numpy as jnp
from jax import lax
from jax.experimental import pallas as pl
from jax.experimental.pallas import tpu as pltpu
from jax.experimental.pallas import tpu_sc as plsc

DMIN, DMAX, STEP = 0.0, 8.0, 0.25
VAR = STEP
NFILT = 33
NC, NS = 2, 16
SUBS_PER_CASE = 8


def _sc_dist2(coords_p, nbr_p, lat_pad, B, N, M, NP):
    NT = NP // 128
    NTW = NT // SUBS_PER_CASE
    LW = NTW * 128
    MT = M // 8
    CASE_STRIDE = MT * NT * 1024
    MT_STRIDE = NT * 1024
    mesh = plsc.VectorSubcoreMesh(
        core_axis_name="c", subcore_axis_name="s", num_cores=NC, num_subcores=NS
    )

    @functools.partial(
        pl.kernel,
        mesh=mesh,
        compiler_params=pltpu.CompilerParams(needs_layout_passes=False),
        out_type=jax.ShapeDtypeStruct((2 * B * CASE_STRIDE,), jnp.float32),
        scratch_types=[
            pltpu.VMEM((NP,), jnp.float32),
            pltpu.VMEM((NP,), jnp.float32),
            pltpu.VMEM((NP,), jnp.float32),
            pltpu.VMEM((M * LW,), jnp.int32),
            pltpu.VMEM((MT * NTW * 1024,), jnp.float32),
            pltpu.VMEM((48,), jnp.float32),
            pltpu.SemaphoreType.DMA,
        ],
    )
    def k(coords_hbm, nbr_hbm, lat_hbm, out, xv, yv, zv, nb, ov, latv, sem):
        c = lax.axis_index("c")
        s = lax.axis_index("s")
        b = s // SUBS_PER_CASE
        sub = s % SUBS_PER_CASE
        nt0 = sub * NTW
        case = c * B + b

        copies = []
        cbase = (b * 3) * 2 + c
        copies.append(pltpu.make_async_copy(
            coords_hbm.at[pl.ds((cbase + 0) * NP, NP)], xv, sem))
        copies.append(pltpu.make_async_copy(
            coords_hbm.at[pl.ds((cbase + 2) * NP, NP)], yv, sem))
        copies.append(pltpu.make_async_copy(
            coords_hbm.at[pl.ds((cbase + 4) * NP, NP)], zv, sem))
        for m in range(M):
            off = ((b * M + m) * 2 + c) * NP + nt0 * 128
            copies.append(pltpu.make_async_copy(
                nbr_hbm.at[pl.ds(off, LW)], nb.at[pl.ds(m * LW, LW)], sem))
        for cp in copies:
            cp.start()
        pltpu.sync_copy(lat_hbm.at[pl.ds(case * 48, 48)], latv)
        lxv = latv[pl.ds(0, 16)]
        lyv = latv[pl.ds(16, 16)]
        lzv = latv[pl.ds(32, 16)]
        for cp in copies:
            cp.wait()

        def pbody(p, _):
            base = p * 16
            ax = xv[pl.ds(nt0 * 128 + base, 16)]
            ay = yv[pl.ds(nt0 * 128 + base, 16)]
            az = zv[pl.ds(nt0 * 128 + base, 16)]
            t = base // 128
            j = base - t * 128
            odst = t * 1024 + j

            @plsc.parallel_loop(0, M, unroll=4)
            def mbody(m):
                idx = nb[pl.ds(m * LW + base, 16)]
                adx = jnp.abs(ax - plsc.load_gather(xv, [idx]))
                ady = jnp.abs(ay - plsc.load_gather(yv, [idx]))
                adz = jnp.abs(az - plsc.load_gather(zv, [idx]))
                ex = jnp.minimum(adx, lxv - adx)
                ey = jnp.minimum(ady, lyv - ady)
                ez = jnp.minimum(adz, lzv - adz)
                ov[pl.ds((m // 8) * (NTW * 1024) + (m % 8) * 128 + odst, 16)] = (
                    ex * ex + ey * ey + ez * ez
                )

            return 0

        lax.fori_loop(0, LW // 16, pbody, 0)

        for mt in range(MT):
            pltpu.sync_copy(
                ov.at[pl.ds(mt * NTW * 1024, NTW * 1024)],
                out.at[pl.ds(case * CASE_STRIDE + mt * MT_STRIDE + nt0 * 1024,
                             NTW * 1024)],
            )

    return k(coords_p, nbr_p, lat_pad)


def _tc_expand(d2v, stack, B, N, M, NP):
    MT = M // 8
    NT = NP // 128
    NTB = 10 if NT % 10 == 0 else (8 if NT % 8 == 0 else 1)
    NL = NTB * 128
    nblk = NT // NTB
    inv_var2 = 1.0 / (VAR * VAR)

    def body(d2_ref, o_ref):
        d = jnp.sqrt(d2_ref[...])
        for mt in range(MT):
            for t in range(NTB):
                dv = d[0, mt, t]
                for k in range(NFILT):
                    diff = dv - (k * STEP)
                    o_ref[0, k, mt * 8:(mt + 1) * 8, t * 128:(t + 1) * 128] = (
                        jnp.exp(diff * diff * (-inv_var2))
                    )

    return pl.pallas_call(
        body,
        grid=(B, nblk),
        in_specs=[
            pl.BlockSpec((1, MT, NTB, 8, 128),
                         lambda b, t: (stack * B + b, 0, t, 0, 0)),
        ],
        out_specs=pl.BlockSpec((1, NFILT, M, NL), lambda b, t: (b, 0, 0, t)),
        out_shape=jax.ShapeDtypeStruct((B, NFILT, M, N), jnp.float32),
    )(d2v)


def kernel(stacked_coords, stacked_lattices, stacked_nbr_lists):
    B, N = stacked_coords.shape[0], stacked_coords.shape[1]
    M = stacked_nbr_lists.shape[2]
    NP = -(-N // (SUBS_PER_CASE * 128)) * (SUBS_PER_CASE * 128)

    nbr1 = stacked_nbr_lists[..., 0]
    nbr2 = stacked_nbr_lists[..., 1]

    coords_p = jnp.pad(
        jnp.transpose(stacked_coords, (0, 2, 3, 1)),
        ((0, 0), (0, 0), (0, 0), (0, NP - N)),
    ).reshape(-1)
    nbr_p = jnp.pad(
        jnp.transpose(stacked_nbr_lists, (0, 2, 3, 1)),
        ((0, 0), (0, 0), (0, 0), (0, NP - N)),
    ).reshape(-1)
    lat_t = jnp.transpose(stacked_lattices, (2, 0, 1))
    lat_pad = jnp.broadcast_to(lat_t[..., None], (2, B, 3, 16)).reshape(-1)

    d2f = _sc_dist2(coords_p, nbr_p, lat_pad, B, N, M, NP)
    d2v = d2f.reshape(2 * B, M // 8, NP // 128, 8, 128)

    bf1 = _tc_expand(d2v, 0, B, N, M, NP)
    bf2 = _tc_expand(d2v, 1, B, N, M, NP)
    bond_fea_1 = jnp.transpose(bf1, (0, 3, 2, 1))
    bond_fea_2 = jnp.transpose(bf2, (0, 3, 2, 1))
    return (nbr1, bond_fea_1, nbr2, bond_fea_2)


# speedup vs baseline: 11.9486x; 1.0039x over previous
import functools

import jax
import jax.numpy as jnp
from jax import lax
from jax.experimental import pallas as pl
from jax.experimental.pallas import tpu as pltpu
from jax.experimental.pallas import tpu_sc as plsc

DMIN, DMAX, STEP = 0.0, 8.0, 0.25
VAR = STEP
NFILT = 33
NC, NS = 2, 16
SUBS_PER_CASE = 8


def _sc_dist2(coords_p, nbr_p, lat_pad, stack, B, N, M, NP):
    NT = NP // 128
    NTW = NT // NS
    LW = NTW * 128
    MT = M // 8
    CASE_STRIDE = MT * NT * 1024
    MT_STRIDE = NT * 1024
    mesh = plsc.VectorSubcoreMesh(
        core_axis_name="c", subcore_axis_name="s", num_cores=NC, num_subcores=NS
    )

    @functools.partial(
        pl.kernel,
        mesh=mesh,
        compiler_params=pltpu.CompilerParams(needs_layout_passes=False),
        out_type=jax.ShapeDtypeStruct((B * CASE_STRIDE,), jnp.float32),
        scratch_types=[
            pltpu.VMEM((NP,), jnp.float32),
            pltpu.VMEM((NP,), jnp.float32),
            pltpu.VMEM((NP,), jnp.float32),
            pltpu.VMEM((M * LW,), jnp.int32),
            pltpu.VMEM((MT * NTW * 1024,), jnp.float32),
            pltpu.VMEM((48,), jnp.float32),
            pltpu.SemaphoreType.DMA,
        ],
    )
    def k(coords_hbm, nbr_hbm, lat_hbm, out, xv, yv, zv, nb, ov, latv, sem):
        b = lax.axis_index("c")
        sub = lax.axis_index("s")
        nt0 = sub * NTW
        case = stack * B + b

        copies = []
        cbase = (b * 3) * 2 + stack
        copies.append(pltpu.make_async_copy(
            coords_hbm.at[pl.ds((cbase + 0) * NP, NP)], xv, sem))
        copies.append(pltpu.make_async_copy(
            coords_hbm.at[pl.ds((cbase + 2) * NP, NP)], yv, sem))
        copies.append(pltpu.make_async_copy(
            coords_hbm.at[pl.ds((cbase + 4) * NP, NP)], zv, sem))
        for m in range(M):
            off = ((b * M + m) * 2 + stack) * NP + nt0 * 128
            copies.append(pltpu.make_async_copy(
                nbr_hbm.at[pl.ds(off, LW)], nb.at[pl.ds(m * LW, LW)], sem))
        for cp in copies:
            cp.start()
        pltpu.sync_copy(lat_hbm.at[pl.ds(case * 48, 48)], latv)
        lxv = latv[pl.ds(0, 16)]
        lyv = latv[pl.ds(16, 16)]
        lzv = latv[pl.ds(32, 16)]
        for cp in copies:
            cp.wait()

        def pbody(p, _):
            base = p * 16
            ax = xv[pl.ds(nt0 * 128 + base, 16)]
            ay = yv[pl.ds(nt0 * 128 + base, 16)]
            az = zv[pl.ds(nt0 * 128 + base, 16)]
            t = base // 128
            j = base - t * 128
            odst = t * 1024 + j

            @plsc.parallel_loop(0, M, unroll=4)
            def mbody(m):
                idx = nb[pl.ds(m * LW + base, 16)]
                adx = jnp.abs(ax - plsc.load_gather(xv, [idx]))
                ady = jnp.abs(ay - plsc.load_gather(yv, [idx]))
                adz = jnp.abs(az - plsc.load_gather(zv, [idx]))
                ex = jnp.minimum(adx, lxv - adx)
                ey = jnp.minimum(ady, lyv - ady)
                ez = jnp.minimum(adz, lzv - adz)
                ov[pl.ds((m // 8) * (NTW * 1024) + (m % 8) * 128 + odst, 16)] = (
                    ex * ex + ey * ey + ez * ez
                )

            return 0

        lax.fori_loop(0, LW // 16, pbody, 0)

        for mt in range(MT):
            pltpu.sync_copy(
                ov.at[pl.ds(mt * NTW * 1024, NTW * 1024)],
                out.at[pl.ds(b * CASE_STRIDE + mt * MT_STRIDE + nt0 * 1024,
                             NTW * 1024)],
            )

    return k(coords_p, nbr_p, lat_pad)


def _tc_expand(d2v, B, N, M, NP):
    MT = M // 8
    NT = NP // 128
    NTB = 10 if NT % 10 == 0 else (8 if NT % 8 == 0 else 1)
    NL = NTB * 128
    nblk = NT // NTB
    inv_var2 = 1.0 / (VAR * VAR)

    def body(d2_ref, o_ref):
        d = jnp.sqrt(d2_ref[...])
        for mt in range(MT):
            for t in range(NTB):
                dv = d[0, mt, t]
                for k in range(NFILT):
                    diff = dv - (k * STEP)
                    o_ref[0, k, mt * 8:(mt + 1) * 8, t * 128:(t + 1) * 128] = (
                        jnp.exp(diff * diff * (-inv_var2))
                    )

    return pl.pallas_call(
        body,
        grid=(B, nblk),
        in_specs=[
            pl.BlockSpec((1, MT, NTB, 8, 128),
                         lambda b, t: (b, 0, t, 0, 0)),
        ],
        out_specs=pl.BlockSpec((1, NFILT, M, NL), lambda b, t: (b, 0, 0, t)),
        out_shape=jax.ShapeDtypeStruct((B, NFILT, M, N), jnp.float32),
    )(d2v)


def kernel(stacked_coords, stacked_lattices, stacked_nbr_lists):
    B, N = stacked_coords.shape[0], stacked_coords.shape[1]
    M = stacked_nbr_lists.shape[2]
    NP = -(-N // (SUBS_PER_CASE * 128)) * (SUBS_PER_CASE * 128)

    nbr1 = stacked_nbr_lists[..., 0]
    nbr2 = stacked_nbr_lists[..., 1]

    coords_p = jnp.pad(
        jnp.transpose(stacked_coords, (0, 2, 3, 1)),
        ((0, 0), (0, 0), (0, 0), (0, NP - N)),
    ).reshape(-1)
    nbr_p = jnp.pad(
        jnp.transpose(stacked_nbr_lists, (0, 2, 3, 1)),
        ((0, 0), (0, 0), (0, 0), (0, NP - N)),
    ).reshape(-1)
    lat_t = jnp.transpose(stacked_lattices, (2, 0, 1))
    lat_pad = jnp.broadcast_to(lat_t[..., None], (2, B, 3, 16)).reshape(-1)

    d2f1 = _sc_dist2(coords_p, nbr_p, lat_pad, 0, B, N, M, NP)
    d2f2 = _sc_dist2(coords_p, nbr_p, lat_pad, 1, B, N, M, NP)
    shape5 = (B, M // 8, NP // 128, 8, 128)
    bf1 = _tc_expand(d2f1.reshape(shape5), B, N, M, NP)
    bf2 = _tc_expand(d2f2.reshape(shape5), B, N, M, NP)
    bond_fea_1 = jnp.transpose(bf1, (0, 3, 2, 1))
    bond_fea_2 = jnp.transpose(bf2, (0, 3, 2, 1))
    return (nbr1, bond_fea_1, nbr2, bond_fea_2)
